# Initial kernel scaffold; baseline (speedup 1.0000x reference)
import functools

import jax
import jax.numpy as jnp
from jax import lax
from jax.experimental import pallas as pl
from jax.experimental.pallas import tpu as pltpu
from jax.experimental.pallas import tpu_sc as plsc

N = 320000
D = 128
NUM_SEGMENTS = 10000

NC = 2
NS = 16
NW = NC * NS
ROWS_PER_W = N // NW
CHUNK = 80
NCHUNK = ROWS_PER_W // CHUNK
SEG_PER_T = NUM_SEGMENTS // NS
STAGE = 125
NSTAGE = SEG_PER_T // STAGE
CW = 16


def _sc_body(x_hbm, b_hbm, sums_out, cnts_out,
             xbuf, ibuf, ones, zbuf, cbuf, acc, cacc):
    c = lax.axis_index("c")
    s = lax.axis_index("s")
    wid = c * NS + s

    one16 = jnp.full((16,), 1.0, dtype=jnp.float32)
    zero16 = jnp.zeros((16,), dtype=jnp.float32)

    def fill_ones(i, _):
        ones[i, :] = one16
        return 0
    lax.fori_loop(0, CHUNK, fill_ones, 0)

    def fill_zrow(i, _):
        def fill_zcol(j, _):
            zbuf[i, pl.ds(j * 16, 16)] = zero16
            return 0
        lax.fori_loop(0, D // 16, fill_zcol, 0)
        cbuf[i, :] = zero16
        return 0
    lax.fori_loop(0, STAGE, fill_zrow, 0)

    def zero_slice(t, _):
        pltpu.sync_copy(zbuf, acc.at[pl.ds(s * SEG_PER_T + t * STAGE, STAGE)])
        pltpu.sync_copy(cbuf, cacc.at[pl.ds(s * SEG_PER_T + t * STAGE, STAGE)])
        return 0
    lax.fori_loop(0, NSTAGE, zero_slice, 0)
    plsc.subcore_barrier()

    base0 = wid * ROWS_PER_W

    def chunk_step(j, _):
        base = base0 + j * CHUNK
        pltpu.sync_copy(x_hbm.at[pl.ds(base, CHUNK)], xbuf)
        pltpu.sync_copy(b_hbm.at[pl.ds(base, CHUNK)], ibuf)
        pltpu.sync_copy(xbuf, acc.at[ibuf], add=True)
        pltpu.sync_copy(ones, cacc.at[ibuf], add=True)
        return 0
    lax.fori_loop(0, NCHUNK, chunk_step, 0)

    plsc.subcore_barrier()

    out_base = c * NUM_SEGMENTS + s * SEG_PER_T

    def stage_step(t, _):
        src = s * SEG_PER_T + t * STAGE
        pltpu.sync_copy(acc.at[pl.ds(src, STAGE)], zbuf)
        pltpu.sync_copy(zbuf, sums_out.at[pl.ds(out_base + t * STAGE, STAGE)])
        pltpu.sync_copy(cacc.at[pl.ds(src, STAGE)], cbuf)
        pltpu.sync_copy(cbuf, cnts_out.at[pl.ds(out_base + t * STAGE, STAGE)])
        return 0
    lax.fori_loop(0, NSTAGE, stage_step, 0)


_sc_scatter = functools.partial(
    pl.kernel,
    out_type=(
        jax.ShapeDtypeStruct((NC * NUM_SEGMENTS, D), jnp.float32),
        jax.ShapeDtypeStruct((NC * NUM_SEGMENTS, CW), jnp.float32),
    ),
    mesh=plsc.VectorSubcoreMesh(core_axis_name="c", subcore_axis_name="s"),
    scratch_types=[
        pltpu.VMEM((CHUNK, D), jnp.float32),
        pltpu.VMEM((CHUNK,), jnp.int32),
        pltpu.VMEM((CHUNK, CW), jnp.float32),
        pltpu.VMEM((STAGE, D), jnp.float32),
        pltpu.VMEM((STAGE, CW), jnp.float32),
        pltpu.VMEM_SHARED((NUM_SEGMENTS, D), jnp.float32),
        pltpu.VMEM_SHARED((NUM_SEGMENTS, CW), jnp.float32),
    ],
)(_sc_body)


BS = 400


def _combine_body(s_ref, c_ref, o_ref):
    sums = s_ref[0] + s_ref[1]
    cnt = c_ref[0, :, 0:1] + c_ref[1, :, 0:1]
    o_ref[...] = sums / jnp.maximum(cnt, 1.0)


def kernel(x, batch):
    sums, cnts = _sc_scatter(x, batch)
    sums = sums.reshape(NC, NUM_SEGMENTS, D)
    cnts = cnts.reshape(NC, NUM_SEGMENTS, CW)
    out = pl.pallas_call(
        _combine_body,
        out_shape=jax.ShapeDtypeStruct((NUM_SEGMENTS, D), jnp.float32),
        grid=(NUM_SEGMENTS // BS,),
        in_specs=[
            pl.BlockSpec((NC, BS, D), lambda i: (0, i, 0)),
            pl.BlockSpec((NC, BS, CW), lambda i: (0, i, 0)),
        ],
        out_specs=pl.BlockSpec((BS, D), lambda i: (i, 0)),
    )(sums, cnts)
    return out


# speedup vs baseline: 3.4789x; 3.4789x over previous
import functools

import jax
import jax.numpy as jnp
from jax import lax
from jax.experimental import pallas as pl
from jax.experimental.pallas import tpu as pltpu
from jax.experimental.pallas import tpu_sc as plsc

N = 320000
D = 128
NUM_SEGMENTS = 10000

NC = 2
NS = 16
NW = NC * NS
ROWS_PER_W = N // NW
CHUNK = 80
NCHUNK = ROWS_PER_W // CHUNK
SEG_PAD = 10240
SEG_PER_T = SEG_PAD // NS
STAGE = 128
NSTAGE = SEG_PER_T // STAGE


def _sums_body(x_hbm, b_hbm, sums_out, xbuf, ibuf, zbuf, acc):
    c = lax.axis_index("c")
    s = lax.axis_index("s")
    wid = c * NS + s

    zero16 = jnp.zeros((16,), dtype=jnp.float32)

    def fill_zrow(i, _):
        def fill_zcol(j, _):
            zbuf[i, pl.ds(j * 16, 16)] = zero16
            return 0
        lax.fori_loop(0, D // 16, fill_zcol, 0)
        return 0
    lax.fori_loop(0, STAGE, fill_zrow, 0)

    def zero_slice(t, _):
        pltpu.sync_copy(zbuf, acc.at[pl.ds(s * SEG_PER_T + t * STAGE, STAGE)])
        return 0
    lax.fori_loop(0, NSTAGE, zero_slice, 0)
    plsc.subcore_barrier()

    base0 = wid * ROWS_PER_W

    def chunk_step(j, _):
        base = base0 + j * CHUNK
        pltpu.sync_copy(x_hbm.at[pl.ds(base, CHUNK)], xbuf)
        pltpu.sync_copy(b_hbm.at[pl.ds(base, CHUNK)], ibuf)
        pltpu.sync_copy(xbuf, acc.at[ibuf], add=True)
        return 0
    lax.fori_loop(0, NCHUNK, chunk_step, 0)

    plsc.subcore_barrier()

    out_base = c * SEG_PAD + s * SEG_PER_T

    def stage_step(t, _):
        src = s * SEG_PER_T + t * STAGE
        pltpu.sync_copy(acc.at[pl.ds(src, STAGE)], zbuf)
        pltpu.sync_copy(zbuf, sums_out.at[pl.ds(out_base + t * STAGE, STAGE)])
        return 0
    lax.fori_loop(0, NSTAGE, stage_step, 0)


def _cnts_body(b_hbm, cnts_out, ibuf, ones, zbuf, cacc):
    c = lax.axis_index("c")
    s = lax.axis_index("s")
    wid = c * NS + s

    zero16 = jnp.zeros((16,), dtype=jnp.float32)
    one16 = jnp.full((16,), 1.0, dtype=jnp.float32)

    def fill_zrow(i, _):
        def fill_zcol(j, _):
            zbuf[i, pl.ds(j * 16, 16)] = zero16
            return 0
        lax.fori_loop(0, D // 16, fill_zcol, 0)
        return 0
    lax.fori_loop(0, STAGE, fill_zrow, 0)

    def fill_orow(i, _):
        def fill_ocol(j, _):
            ones[i, pl.ds(j * 16, 16)] = one16
            return 0
        lax.fori_loop(0, D // 16, fill_ocol, 0)
        return 0
    lax.fori_loop(0, CHUNK, fill_orow, 0)

    def zero_slice(t, _):
        pltpu.sync_copy(zbuf, cacc.at[pl.ds(s * SEG_PER_T + t * STAGE, STAGE)])
        return 0
    lax.fori_loop(0, NSTAGE, zero_slice, 0)
    plsc.subcore_barrier()

    base0 = wid * ROWS_PER_W

    def chunk_step(j, _):
        base = base0 + j * CHUNK
        pltpu.sync_copy(b_hbm.at[pl.ds(base, CHUNK)], ibuf)
        pltpu.sync_copy(ones, cacc.at[ibuf], add=True)
        return 0
    lax.fori_loop(0, NCHUNK, chunk_step, 0)

    plsc.subcore_barrier()

    out_base = c * SEG_PAD + s * SEG_PER_T

    def stage_step(t, _):
        src = s * SEG_PER_T + t * STAGE
        pltpu.sync_copy(cacc.at[pl.ds(src, STAGE)], zbuf)
        pltpu.sync_copy(zbuf, cnts_out.at[pl.ds(out_base + t * STAGE, STAGE)])
        return 0
    lax.fori_loop(0, NSTAGE, stage_step, 0)


_sc_sums = functools.partial(
    pl.kernel,
    out_type=jax.ShapeDtypeStruct((NC * SEG_PAD, D), jnp.float32),
    mesh=plsc.VectorSubcoreMesh(core_axis_name="c", subcore_axis_name="s"),
    scratch_types=[
        pltpu.VMEM((CHUNK, D), jnp.float32),
        pltpu.VMEM((CHUNK,), jnp.int32),
        pltpu.VMEM((STAGE, D), jnp.float32),
        pltpu.VMEM_SHARED((SEG_PAD, D), jnp.float32),
    ],
)(_sums_body)

_sc_cnts = functools.partial(
    pl.kernel,
    out_type=jax.ShapeDtypeStruct((NC * SEG_PAD, D), jnp.float32),
    mesh=plsc.VectorSubcoreMesh(core_axis_name="c", subcore_axis_name="s"),
    scratch_types=[
        pltpu.VMEM((CHUNK,), jnp.int32),
        pltpu.VMEM((CHUNK, D), jnp.float32),
        pltpu.VMEM((STAGE, D), jnp.float32),
        pltpu.VMEM_SHARED((SEG_PAD, D), jnp.float32),
    ],
)(_cnts_body)


BS = 1024


def _combine_body(s_ref, c_ref, o_ref):
    sums = s_ref[0] + s_ref[1]
    cnt = c_ref[0, :, 0:1] + c_ref[1, :, 0:1]
    o_ref[...] = sums / jnp.maximum(cnt, 1.0)


def kernel(x, batch):
    sums = _sc_sums(x, batch)
    cnts = _sc_cnts(batch)
    sums = sums.reshape(NC, SEG_PAD, D)
    cnts = cnts.reshape(NC, SEG_PAD, D)
    out = pl.pallas_call(
        _combine_body,
        out_shape=jax.ShapeDtypeStruct((NUM_SEGMENTS, D), jnp.float32),
        grid=(SEG_PAD // BS,),
        in_specs=[
            pl.BlockSpec((NC, BS, D), lambda i: (0, i, 0)),
            pl.BlockSpec((NC, BS, D), lambda i: (0, i, 0)),
        ],
        out_specs=pl.BlockSpec((BS, D), lambda i: (i, 0)),
    )(sums, cnts)
    return out


# speedup vs baseline: 6.8349x; 1.9647x over previous
import functools

import jax
import jax.numpy as jnp
from jax import lax
from jax.experimental import pallas as pl
from jax.experimental.pallas import tpu as pltpu
from jax.experimental.pallas import tpu_sc as plsc

N = 320000
D = 128
NUM_SEGMENTS = 10000

NC = 2
NS = 16
NW = NC * NS
ROWS_PER_W = N // NW
CHUNK = 80
NCHUNK = ROWS_PER_W // CHUNK
NBUF = 3
NOUTER = (NCHUNK + NBUF - 1) // NBUF
SEG_PAD = 10240
SEG_PER_T = SEG_PAD // NS
STAGE = 64
NSTAGE = SEG_PER_T // STAGE


def _sums_body(x_hbm, b_hbm, sums_out,
               xb0, xb1, xb2, ib0, ib1, ib2, zbuf, acc,
               sx0, sx1, sx2, si0, si1, si2, ss0, ss1, ss2):
    c = lax.axis_index("c")
    s = lax.axis_index("s")
    wid = c * NS + s
    base0 = wid * ROWS_PER_W

    xbs = (xb0, xb1, xb2)
    ibs = (ib0, ib1, ib2)
    sxs = (sx0, sx1, sx2)
    sis = (si0, si1, si2)
    sss = (ss0, ss1, ss2)

    def dma_start(j, k):
        base = base0 + j * CHUNK
        pltpu.async_copy(x_hbm.at[pl.ds(base, CHUNK)], xbs[k], sxs[k])
        pltpu.async_copy(b_hbm.at[pl.ds(base, CHUNK)], ibs[k], sis[k])

    def dma_wait(j, k):
        base = base0 + j * CHUNK
        pltpu.make_async_copy(x_hbm.at[pl.ds(base, CHUNK)], xbs[k], sxs[k]).wait()
        pltpu.make_async_copy(b_hbm.at[pl.ds(base, CHUNK)], ibs[k], sis[k]).wait()

    def scat_start(k):
        pltpu.async_copy(xbs[k], acc.at[ibs[k]], sss[k], add=True)

    def scat_wait(k):
        pltpu.make_async_copy(xbs[k], acc.at[ibs[k]], sss[k]).wait()

    dma_start(0, 0)
    dma_start(1, 1)

    zero16 = jnp.zeros((16,), dtype=jnp.float32)

    def fill_zrow(i, _):
        def fill_zcol(j, _):
            zbuf[i, pl.ds(j * 16, 16)] = zero16
            return 0
        lax.fori_loop(0, D // 16, fill_zcol, 0)
        return 0
    lax.fori_loop(0, STAGE, fill_zrow, 0)

    def zero_slice(t, _):
        pltpu.sync_copy(zbuf, acc.at[pl.ds(s * SEG_PER_T + t * STAGE, STAGE)])
        return 0
    lax.fori_loop(0, NSTAGE, zero_slice, 0)
    plsc.subcore_barrier()

    def outer(i, _):
        for k in range(NBUF):
            j = i * NBUF + k
            kp = (k - 1) % NBUF

            def step():
                if k == 0:
                    @pl.when(i >= 1)
                    def _():
                        scat_wait(kp)
                else:
                    scat_wait(kp)

                @pl.when(j + NBUF - 1 < NCHUNK)
                def _():
                    dma_start(j + NBUF - 1, kp)

                dma_wait(j, k)
                scat_start(k)

            if (NOUTER - 1) * NBUF + k >= NCHUNK:
                @pl.when(j < NCHUNK)
                def _():
                    step()
            else:
                step()
        return 0
    lax.fori_loop(0, NOUTER, outer, 0)
    scat_wait((NCHUNK - 1) % NBUF)

    plsc.subcore_barrier()

    out_base = c * SEG_PAD + s * SEG_PER_T

    def stage_step(t, _):
        src = s * SEG_PER_T + t * STAGE
        pltpu.sync_copy(acc.at[pl.ds(src, STAGE)], zbuf)
        pltpu.sync_copy(zbuf, sums_out.at[pl.ds(out_base + t * STAGE, STAGE)])
        return 0
    lax.fori_loop(0, NSTAGE, stage_step, 0)


def _cnts_body(b_hbm, cnts_out,
               ib0, ib1, ib2, ones, zbuf, cacc,
               si0, si1, si2, ss0, ss1, ss2):
    c = lax.axis_index("c")
    s = lax.axis_index("s")
    wid = c * NS + s
    base0 = wid * ROWS_PER_W

    ibs = (ib0, ib1, ib2)
    sis = (si0, si1, si2)
    sss = (ss0, ss1, ss2)

    def dma_start(j, k):
        base = base0 + j * CHUNK
        pltpu.async_copy(b_hbm.at[pl.ds(base, CHUNK)], ibs[k], sis[k])

    def dma_wait(j, k):
        base = base0 + j * CHUNK
        pltpu.make_async_copy(b_hbm.at[pl.ds(base, CHUNK)], ibs[k], sis[k]).wait()

    def scat_start(k):
        pltpu.async_copy(ones, cacc.at[ibs[k]], sss[k], add=True)

    def scat_wait(k):
        pltpu.make_async_copy(ones, cacc.at[ibs[k]], sss[k]).wait()

    dma_start(0, 0)
    dma_start(1, 1)

    zero16 = jnp.zeros((16,), dtype=jnp.float32)
    one16 = jnp.full((16,), 1.0, dtype=jnp.float32)

    def fill_zrow(i, _):
        def fill_zcol(j, _):
            zbuf[i, pl.ds(j * 16, 16)] = zero16
            return 0
        lax.fori_loop(0, D // 16, fill_zcol, 0)
        return 0
    lax.fori_loop(0, STAGE, fill_zrow, 0)

    def fill_orow(i, _):
        def fill_ocol(j, _):
            ones[i, pl.ds(j * 16, 16)] = one16
            return 0
        lax.fori_loop(0, D // 16, fill_ocol, 0)
        return 0
    lax.fori_loop(0, CHUNK, fill_orow, 0)

    def zero_slice(t, _):
        pltpu.sync_copy(zbuf, cacc.at[pl.ds(s * SEG_PER_T + t * STAGE, STAGE)])
        return 0
    lax.fori_loop(0, NSTAGE, zero_slice, 0)
    plsc.subcore_barrier()

    def outer(i, _):
        for k in range(NBUF):
            j = i * NBUF + k
            kp = (k - 1) % NBUF

            def step():
                if k == 0:
                    @pl.when(i >= 1)
                    def _():
                        scat_wait(kp)
                else:
                    scat_wait(kp)

                @pl.when(j + NBUF - 1 < NCHUNK)
                def _():
                    dma_start(j + NBUF - 1, kp)

                dma_wait(j, k)
                scat_start(k)

            if (NOUTER - 1) * NBUF + k >= NCHUNK:
                @pl.when(j < NCHUNK)
                def _():
                    step()
            else:
                step()
        return 0
    lax.fori_loop(0, NOUTER, outer, 0)
    scat_wait((NCHUNK - 1) % NBUF)

    plsc.subcore_barrier()

    out_base = c * SEG_PAD + s * SEG_PER_T

    def stage_step(t, _):
        src = s * SEG_PER_T + t * STAGE
        pltpu.sync_copy(cacc.at[pl.ds(src, STAGE)], zbuf)
        pltpu.sync_copy(zbuf, cnts_out.at[pl.ds(out_base + t * STAGE, STAGE)])
        return 0
    lax.fori_loop(0, NSTAGE, stage_step, 0)


_sc_sums = functools.partial(
    pl.kernel,
    out_type=jax.ShapeDtypeStruct((NC * SEG_PAD, D), jnp.float32),
    mesh=plsc.VectorSubcoreMesh(core_axis_name="c", subcore_axis_name="s"),
    scratch_types=[
        pltpu.VMEM((CHUNK, D), jnp.float32),
        pltpu.VMEM((CHUNK, D), jnp.float32),
        pltpu.VMEM((CHUNK, D), jnp.float32),
        pltpu.VMEM((CHUNK,), jnp.int32),
        pltpu.VMEM((CHUNK,), jnp.int32),
        pltpu.VMEM((CHUNK,), jnp.int32),
        pltpu.VMEM((STAGE, D), jnp.float32),
        pltpu.VMEM_SHARED((SEG_PAD, D), jnp.float32),
        pltpu.SemaphoreType.DMA,
        pltpu.SemaphoreType.DMA,
        pltpu.SemaphoreType.DMA,
        pltpu.SemaphoreType.DMA,
        pltpu.SemaphoreType.DMA,
        pltpu.SemaphoreType.DMA,
        pltpu.SemaphoreType.DMA,
        pltpu.SemaphoreType.DMA,
        pltpu.SemaphoreType.DMA,
    ],
)(_sums_body)

_sc_cnts = functools.partial(
    pl.kernel,
    out_type=jax.ShapeDtypeStruct((NC * SEG_PAD, D), jnp.float32),
    mesh=plsc.VectorSubcoreMesh(core_axis_name="c", subcore_axis_name="s"),
    scratch_types=[
        pltpu.VMEM((CHUNK,), jnp.int32),
        pltpu.VMEM((CHUNK,), jnp.int32),
        pltpu.VMEM((CHUNK,), jnp.int32),
        pltpu.VMEM((CHUNK, D), jnp.float32),
        pltpu.VMEM((STAGE, D), jnp.float32),
        pltpu.VMEM_SHARED((SEG_PAD, D), jnp.float32),
        pltpu.SemaphoreType.DMA,
        pltpu.SemaphoreType.DMA,
        pltpu.SemaphoreType.DMA,
        pltpu.SemaphoreType.DMA,
        pltpu.SemaphoreType.DMA,
        pltpu.SemaphoreType.DMA,
    ],
)(_cnts_body)


BS = 1024


def _combine_body(s_ref, c_ref, o_ref):
    sums = s_ref[0] + s_ref[1]
    cnt = c_ref[0, :, 0:1] + c_ref[1, :, 0:1]
    o_ref[...] = sums / jnp.maximum(cnt, 1.0)


def kernel(x, batch):
    sums = _sc_sums(x, batch)
    cnts = _sc_cnts(batch)
    sums = sums.reshape(NC, SEG_PAD, D)
    cnts = cnts.reshape(NC, SEG_PAD, D)
    out = pl.pallas_call(
        _combine_body,
        out_shape=jax.ShapeDtypeStruct((NUM_SEGMENTS, D), jnp.float32),
        grid=(SEG_PAD // BS,),
        in_specs=[
            pl.BlockSpec((NC, BS, D), lambda i: (0, i, 0)),
            pl.BlockSpec((NC, BS, D), lambda i: (0, i, 0)),
        ],
        out_specs=pl.BlockSpec((BS, D), lambda i: (i, 0)),
    )(sums, cnts)
    return out
